# baseline (speedup 1.0000x reference)
import functools

import jax
import jax.numpy as jnp
from jax import lax
from jax.experimental import pallas as pl
from jax.experimental.pallas import tpu as pltpu
from jax.experimental.pallas import tpu_sc as plsc

SEQ = 50


def _gather_sc(emb, idx):
    (n,) = idx.shape
    d = emb.shape[1]
    nw = 32
    per_w = n // nw
    ch = 80
    nch = per_w // ch
    idx3 = idx.reshape(nw, nch, ch)

    mesh = plsc.VectorSubcoreMesh(core_axis_name="c", subcore_axis_name="s")

    @functools.partial(
        pl.kernel,
        mesh=mesh,
        out_type=jax.ShapeDtypeStruct((n, d), jnp.float32),
        scratch_types=[
            pltpu.VMEM((nch, ch), jnp.int32),
            pltpu.VMEM((ch, d), jnp.float32),
            pltpu.SemaphoreType.DMA,
        ],
    )
    def gath(emb_hbm, idx_hbm, out_hbm, idx_v, rows_v, sem):
        wid = lax.axis_index("s") * 2 + lax.axis_index("c")
        base = wid * per_w
        pltpu.sync_copy(idx_hbm.at[wid], idx_v)
        for j in range(nch):
            pltpu.async_copy(emb_hbm.at[idx_v.at[j]], rows_v, sem).wait()
            pltpu.sync_copy(rows_v, out_hbm.at[pl.ds(base + j * ch, ch)])

    return gath(emb, idx3)


def _mlp_tc(e2d, w1, b1, w2, b2):
    n, e_dim = e2d.shape
    h_dim = w1.shape[1]
    o_dim = w2.shape[1]
    bb = 8
    rows = bb * SEQ
    grid = n // rows

    def body(e_ref, w1_ref, b1_ref, w2_ref, b2_ref, o_ref):
        e = e_ref[...]
        h = jnp.maximum(
            jnp.dot(e, w1_ref[...], preferred_element_type=jnp.float32)
            + b1_ref[...],
            0.0,
        )
        logits = (
            jnp.dot(h, w2_ref[...], preferred_element_type=jnp.float32)
            + b2_ref[...]
        )
        m = jnp.max(logits, axis=0, keepdims=True)
        p = jnp.exp(logits - m)
        at_seg = lax.broadcasted_iota(jnp.int32, (bb, rows), 1) // SEQ
        at_col = lax.broadcasted_iota(jnp.int32, (bb, rows), 0)
        ind_t = (at_seg == at_col).astype(jnp.float32)
        s = jnp.dot(ind_t, p, preferred_element_type=jnp.float32)
        a_seg = lax.broadcasted_iota(jnp.int32, (rows, bb), 0) // SEQ
        a_col = lax.broadcasted_iota(jnp.int32, (rows, bb), 1)
        ind = (a_seg == a_col).astype(jnp.float32)
        o_ref[...] = p * jnp.dot(
            ind, 1.0 / s, preferred_element_type=jnp.float32
        )

    return pl.pallas_call(
        body,
        grid=(grid,),
        in_specs=[
            pl.BlockSpec((rows, e_dim), lambda i: (i, 0)),
            pl.BlockSpec((e_dim, h_dim), lambda i: (0, 0)),
            pl.BlockSpec((1, h_dim), lambda i: (0, 0)),
            pl.BlockSpec((h_dim, o_dim), lambda i: (0, 0)),
            pl.BlockSpec((1, o_dim), lambda i: (0, 0)),
        ],
        out_specs=pl.BlockSpec((rows, o_dim), lambda i: (i, 0)),
        out_shape=jax.ShapeDtypeStruct((n, o_dim), jnp.float32),
    )(e2d, w1, b1.reshape(1, h_dim), w2, b2.reshape(1, o_dim))


def kernel(x, emb, W1, b1, W2, b2):
    b, l = x.shape
    idx = x.reshape(-1).astype(jnp.int32)
    e2d = _gather_sc(emb, idx)
    out2d = _mlp_tc(e2d, W1, b1, W2, b2)
    return out2d.reshape(b, l, W2.shape[1])


# speedup vs baseline: 2.7930x; 2.7930x over previous
import functools

import jax
import jax.numpy as jnp
from jax import lax
from jax.experimental import pallas as pl
from jax.experimental.pallas import tpu as pltpu
from jax.experimental.pallas import tpu_sc as plsc

SEQ = 50


def _gather_sc(emb, idx):
    (n,) = idx.shape
    d = emb.shape[1]
    nw = 32
    per_w = n // nw
    ch = 80
    nch = per_w // ch
    idx3 = idx.reshape(nw, nch, ch)

    mesh = plsc.VectorSubcoreMesh(core_axis_name="c", subcore_axis_name="s")

    @functools.partial(
        pl.kernel,
        mesh=mesh,
        out_type=jax.ShapeDtypeStruct((n, d), jnp.float32),
        scratch_types=[
            pltpu.VMEM((nch, ch), jnp.int32),
            pltpu.VMEM((ch, d), jnp.float32),
            pltpu.SemaphoreType.DMA,
        ],
    )
    def gath(emb_hbm, idx_hbm, out_hbm, idx_v, rows_v, sem):
        wid = lax.axis_index("s") * 2 + lax.axis_index("c")
        base = wid * per_w
        pltpu.sync_copy(idx_hbm.at[wid], idx_v)
        for j in range(nch):
            pltpu.async_copy(emb_hbm.at[idx_v.at[j]], rows_v, sem).wait()
            pltpu.sync_copy(rows_v, out_hbm.at[pl.ds(base + j * ch, ch)])

    return gath(emb, idx3)


def _mlp_tc(e3t, w1t, b1c, w2t, b2c, nb):
    e_dim = e3t.shape[2]
    h_dim = w1t.shape[0]
    o_dim = w2t.shape[0]
    bb = 128
    ob = 200
    kg = nb // bb
    jg = o_dim // ob
    toks = bb * SEQ

    def body(e_ref, w1_ref, b1_ref, w2_ref, b2_ref, o_ref, ht_ref):
        j = pl.program_id(1)

        @pl.when(j == 0)
        def _():
            e = e_ref[...].reshape(toks, e_dim).astype(jnp.bfloat16)
            ht = lax.dot_general(
                w1_ref[...], e, (((1,), (1,)), ((), ())),
                preferred_element_type=jnp.float32,
            )
            ht_ref[...] = jnp.maximum(ht + b1_ref[...], 0.0).astype(
                jnp.bfloat16
            )

        lt = (
            jnp.dot(w2_ref[...], ht_ref[...],
                    preferred_element_type=jnp.float32)
            + b2_ref[...]
        )
        for l in range(SEQ):
            o_ref[l] = lt[:, l * bb:(l + 1) * bb]
        t = o_ref[...]
        m = jnp.max(t, axis=0, keepdims=True)
        p = jnp.exp(t - m)
        s = jnp.sum(p, axis=0, keepdims=True)
        o_ref[...] = p * (1.0 / s)

    return pl.pallas_call(
        body,
        grid=(kg, jg),
        in_specs=[
            pl.BlockSpec((SEQ, bb, e_dim), lambda k, j: (0, k, 0)),
            pl.BlockSpec((h_dim, e_dim), lambda k, j: (0, 0)),
            pl.BlockSpec((h_dim, 1), lambda k, j: (0, 0)),
            pl.BlockSpec((ob, h_dim), lambda k, j: (j, 0)),
            pl.BlockSpec((ob, 1), lambda k, j: (j, 0)),
        ],
        out_specs=pl.BlockSpec((SEQ, ob, bb), lambda k, j: (0, j, k)),
        out_shape=jax.ShapeDtypeStruct((SEQ, o_dim, nb), jnp.float32),
        scratch_shapes=[pltpu.VMEM((h_dim, toks), jnp.bfloat16)],
    )(e3t, w1t, b1c, w2t, b2c)


def kernel(x, emb, W1, b1, W2, b2):
    b, l = x.shape
    h_dim = W1.shape[1]
    o_dim = W2.shape[1]
    idx = x.T.reshape(-1).astype(jnp.int32)
    e2d = _gather_sc(emb, idx)
    e3t = e2d.reshape(l, b, emb.shape[1])
    ot = _mlp_tc(
        e3t,
        W1.T.astype(jnp.bfloat16),
        b1.reshape(h_dim, 1),
        W2.T.astype(jnp.bfloat16),
        b2.reshape(o_dim, 1),
        b,
    )
    return jnp.transpose(ot, (2, 0, 1))
